# baseline (speedup 1.0000x reference)
import functools

import jax
import jax.numpy as jnp
from jax import lax
from jax.experimental import pallas as pl
from jax.experimental.pallas import tpu as pltpu
from jax.experimental.pallas import tpu_sc as plsc

N = 10000
E = 320000
D = 128
F = D // 2
NP = 10240
NC = 2
NS = 16
ROW = 125
EROWS = E // ROW

_MESH = dict(core_axis_name="c", subcore_axis_name="s")


EPT = E // (NC * NS)


def _deg_body(dst_hbm, ones_hbm, out_hbm, idx_v, ones_v, zrow_v, hist_sh):
    c = lax.axis_index("c")
    s = lax.axis_index("s")

    zero16 = jnp.zeros((16,), jnp.float32)
    for i in range(40):
        zrow_v[pl.ds(i * 16, 16)] = zero16

    pltpu.sync_copy(zrow_v, hist_sh.at[pl.ds(s * 640, 640)])
    pltpu.sync_copy(ones_hbm, ones_v)
    plsc.subcore_barrier()

    w = c * NS + s
    pltpu.sync_copy(dst_hbm.at[w], idx_v)

    pltpu.sync_copy(ones_v, hist_sh.at[idx_v], add=True)
    plsc.subcore_barrier()
    pltpu.sync_copy(hist_sh.at[pl.ds(s * 640, 640)],
                    out_hbm.at[c, pl.ds(s * 640, 640)])


_deg_call = functools.partial(
    pl.kernel,
    out_type=jax.ShapeDtypeStruct((NC, NP), jnp.float32),
    mesh=plsc.VectorSubcoreMesh(**_MESH),
    scratch_types=[
        pltpu.VMEM((EPT,), jnp.int32),
        pltpu.VMEM((EPT,), jnp.float32),
        pltpu.VMEM((640,), jnp.float32),
        pltpu.VMEM_SHARED((NP,), jnp.float32),
    ],
)(_deg_body)


LA = 80


def _agg_body(g_hbm, src3_hbm, dst3_hbm, out_hbm,
              idxs_v, idxd_v, rows_v, agg_sh, gsem, ssem):
    c = lax.axis_index("c")
    s = lax.axis_index("s")

    zero16 = jnp.zeros((16,), jnp.float32)

    def zrow(i, _):
        for k in range(8):
            rows_v[i, pl.ds(k * 16, 16)] = zero16
        return 0

    lax.fori_loop(0, 128, zrow, 0)
    for m in range(5):
        pltpu.sync_copy(rows_v.at[pl.ds(0, 128)],
                        agg_sh.at[pl.ds(s * 640 + m * 128, 128)])
    plsc.subcore_barrier()

    w = c * NS + s

    def gath(j, b):
        return pltpu.make_async_copy(
            g_hbm.at[idxs_v.at[pl.ds(j * LA, LA)]],
            rows_v.at[pl.ds(b * LA, LA)],
            gsem.at[b],
        )

    def scat(j, b):
        return pltpu.make_async_copy(
            rows_v.at[pl.ds(b * LA, LA)],
            agg_sh.at[idxd_v.at[pl.ds(j * LA, LA)]],
            ssem.at[b],
        )

    for h in range(5):
        pltpu.sync_copy(src3_hbm.at[w * 5 + h], idxs_v)
        pltpu.sync_copy(dst3_hbm.at[w * 5 + h], idxd_v)
        gath(0, 0).start()
        gath(1, 1).start()
        for j in range(25):
            b = j % 4
            gath(j, b).wait()
            scat(j, b).start(add=True)
            nj = j + 2
            if nj <= 24:
                if j >= 2:
                    scat(j - 2, nj % 4).wait()
                gath(nj, nj % 4).start()
        for j in (21, 22, 23, 24):
            scat(j, j % 4).wait()

    plsc.subcore_barrier()
    pltpu.sync_copy(agg_sh.at[pl.ds(s * 640, 640)],
                    out_hbm.at[c, pl.ds(s * 640, 640)])


_agg_call = functools.partial(
    pl.kernel,
    out_type=jax.ShapeDtypeStruct((NC, NP, D), jnp.float32),
    mesh=plsc.VectorSubcoreMesh(**_MESH),
    scratch_types=[
        pltpu.VMEM((2000,), jnp.int32),
        pltpu.VMEM((2000,), jnp.int32),
        pltpu.VMEM((4 * LA, D), jnp.float32),
        pltpu.VMEM_SHARED((NP, D), jnp.float32),
        pltpu.SemaphoreType.DMA((4,)),
        pltpu.SemaphoreType.DMA((4,)),
    ],
)(_agg_body)


def _edge_body(s1_hbm, s2_hbm, src_hbm, dst_hbm, out_hbm,
               s1_v, s2_v, srcb_v, dstb_v, outb_v):
    c = lax.axis_index("c")
    s = lax.axis_index("s")
    w = c * NS + s
    base = w * (E // (NC * NS))

    pltpu.sync_copy(s1_hbm, s1_v)
    pltpu.sync_copy(s2_hbm, s2_v)
    pltpu.sync_copy(src_hbm.at[pl.ds(base, 10000)], srcb_v)
    pltpu.sync_copy(dst_hbm.at[pl.ds(base, 10000)], dstb_v)

    def step(k, _):
        i1 = srcb_v[pl.ds(k * 16, 16)]
        i2 = dstb_v[pl.ds(k * 16, 16)]
        a = plsc.load_gather(s1_v, [i1])
        bb = plsc.load_gather(s2_v, [i2])
        outb_v[pl.ds(k * 16, 16)] = a + bb
        return 0

    lax.fori_loop(0, 625, step, 0)
    pltpu.sync_copy(outb_v, out_hbm.at[pl.ds(base, 10000)])


_edge_call = functools.partial(
    pl.kernel,
    out_type=jax.ShapeDtypeStruct((E,), jnp.float32),
    mesh=plsc.VectorSubcoreMesh(**_MESH),
    compiler_params=pltpu.CompilerParams(needs_layout_passes=False),
    scratch_types=[
        pltpu.VMEM((N,), jnp.float32),
        pltpu.VMEM((N,), jnp.float32),
        pltpu.VMEM((10000,), jnp.int32),
        pltpu.VMEM((10000,), jnp.int32),
        pltpu.VMEM((10000,), jnp.float32),
    ],
)(_edge_body)


def _mm_body(x_ref, w_ref, degp_ref, g_ref):
    i = pl.program_id(0)
    deg = degp_ref[0, pl.ds(i * 1280, 1280)] + degp_ref[1, pl.ds(i * 1280, 1280)]
    isd = lax.rsqrt(deg + 1.0)
    h = jnp.dot(x_ref[...], w_ref[...], preferred_element_type=jnp.float32)
    g_ref[...] = h * isd[:, None]


def _mm_call(x, w, degp):
    return pl.pallas_call(
        _mm_body,
        grid=(NP // 1280,),
        in_specs=[
            pl.BlockSpec((1280, D), lambda i: (i, 0)),
            pl.BlockSpec((D, D), lambda i: (0, 0)),
            pl.BlockSpec((NC, NP), lambda i: (0, 0)),
        ],
        out_specs=pl.BlockSpec((1280, D), lambda i: (i, 0)),
        out_shape=jax.ShapeDtypeStruct((N, D), jnp.float32),
    )(x, w, degp)


def _head_body(agg_ref, degp_ref, b_ref, wep_ref, bev_ref, out_ref):
    i = pl.program_id(0)
    deg = degp_ref[0, pl.ds(i * 1280, 1280)] + degp_ref[1, pl.ds(i * 1280, 1280)]
    isd = lax.rsqrt(deg + 1.0)
    agg = agg_ref[0] + agg_ref[1]
    h_out = jnp.maximum(agg * isd[:, None] + b_ref[0], 0.0)
    out_ref[...] = (
        jnp.dot(h_out, wep_ref[...], preferred_element_type=jnp.float32)
        + bev_ref[0]
    )


def _head_call(agg, degp, b2, wep, bev):
    return pl.pallas_call(
        _head_body,
        grid=(NP // 1280,),
        in_specs=[
            pl.BlockSpec((NC, 1280, D), lambda i: (0, i, 0)),
            pl.BlockSpec((NC, NP), lambda i: (0, 0)),
            pl.BlockSpec((1, D), lambda i: (0, 0)),
            pl.BlockSpec((D, D), lambda i: (0, 0)),
            pl.BlockSpec((1, D), lambda i: (0, 0)),
        ],
        out_specs=pl.BlockSpec((1280, D), lambda i: (i, 0)),
        out_shape=jax.ShapeDtypeStruct((N, D), jnp.float32),
    )(agg, degp, b2, wep, bev)


def kernel(x, edge_index, W, b, We, be):
    src = edge_index[0].astype(jnp.int32)
    dst = edge_index[1].astype(jnp.int32)

    srcw = src.reshape(NC * NS, EPT)
    dstw = dst.reshape(NC * NS, EPT)
    srcc = src.reshape(NC * NS * 5, EPT // 5)
    dstc = dst.reshape(NC * NS * 5, EPT // 5)
    ones1d = jnp.ones((EPT,), jnp.float32)
    degp = _deg_call(dstw, ones1d)
    g = _mm_call(x, W, degp)
    agg = _agg_call(g, srcc, dstc)

    wep = jnp.zeros((D, D), jnp.float32)
    wep = wep.at[:, 0].set(We[:D, 0]).at[:, 1].set(We[D:, 0])
    bev = jnp.zeros((1, D), jnp.float32).at[0, 0].set(be[0])
    sout = _head_call(agg, degp, b.reshape(1, D), wep, bev)
    s1 = sout[:, 0]
    s2 = sout[:, 1]

    eout = _edge_call(s1, s2, src, dst)
    return eout[:, None]


# speedup vs baseline: 1.1931x; 1.1931x over previous
import functools

import jax
import jax.numpy as jnp
from jax import lax
from jax.experimental import pallas as pl
from jax.experimental.pallas import tpu as pltpu
from jax.experimental.pallas import tpu_sc as plsc

N = 10000
E = 320000
D = 128
NP = 10240
NC = 2
NS = 16

_MESH = dict(core_axis_name="c", subcore_axis_name="s")


EPT = E // (NC * NS)


def _deg_body(ei_hbm, ones_hbm, out_hbm, idx_v, ones_v, zrow_v, hist_sh):
    c = lax.axis_index("c")
    s = lax.axis_index("s")

    zero16 = jnp.zeros((16,), jnp.float32)
    for i in range(40):
        zrow_v[pl.ds(i * 16, 16)] = zero16

    pltpu.sync_copy(zrow_v, hist_sh.at[pl.ds(s * 640, 640)])
    pltpu.sync_copy(ones_hbm, ones_v)
    plsc.subcore_barrier()

    w = c * NS + s
    pltpu.sync_copy(ei_hbm.at[32 + w], idx_v)

    pltpu.sync_copy(ones_v, hist_sh.at[idx_v], add=True)
    plsc.subcore_barrier()
    pltpu.sync_copy(hist_sh.at[pl.ds(s * 640, 640)],
                    out_hbm.at[c, pl.ds(s * 640, 640)])


_deg_call = functools.partial(
    pl.kernel,
    out_type=jax.ShapeDtypeStruct((NC, NP), jnp.float32),
    mesh=plsc.VectorSubcoreMesh(**_MESH),
    scratch_types=[
        pltpu.VMEM((EPT,), jnp.int32),
        pltpu.VMEM((EPT,), jnp.float32),
        pltpu.VMEM((640,), jnp.float32),
        pltpu.VMEM_SHARED((NP,), jnp.float32),
    ],
)(_deg_body)


LB = 112


def _agg_body(g_hbm, ei_hbm, out_hbm, idxs_v, idxd_v, rows_v, agg_sh, sem):
    c = lax.axis_index("c")
    s = lax.axis_index("s")

    zero16 = jnp.zeros((16,), jnp.float32)

    def zrow(i, _):
        for k in range(8):
            rows_v[i, pl.ds(k * 16, 16)] = zero16
        return 0

    lax.fori_loop(0, 128, zrow, 0)
    for m in range(5):
        pltpu.sync_copy(rows_v.at[pl.ds(0, 128)],
                        agg_sh.at[pl.ds(s * 640 + m * 128, 128)])
    plsc.subcore_barrier()

    w = c * NS + s
    pltpu.sync_copy(ei_hbm.at[w], idxs_v)
    pltpu.sync_copy(ei_hbm.at[32 + w], idxd_v)

    def gath(off, ln, b):
        return pltpu.make_async_copy(
            g_hbm.at[idxs_v.at[pl.ds(off, ln)]],
            rows_v.at[pl.ds(b * LB, ln)],
            sem.at[b],
        )

    def scat(off, ln, b):
        pltpu.sync_copy(rows_v.at[pl.ds(b * LB, ln)],
                        agg_sh.at[idxd_v.at[pl.ds(off, ln)]], add=True)

    spans = [(j * LB, LB) for j in range(89)] + [(89 * LB, EPT - 89 * LB)]
    gath(*spans[0], 0).start()
    gath(*spans[1], 1).start()
    for j in range(90):
        b = j % 2
        gath(*spans[j], b).wait()
        scat(*spans[j], b)
        if j + 2 <= 89:
            gath(*spans[j + 2], b).start()

    plsc.subcore_barrier()
    pltpu.sync_copy(agg_sh.at[pl.ds(s * 640, 640)],
                    out_hbm.at[c, pl.ds(s * 640, 640)])


_agg_call = functools.partial(
    pl.kernel,
    out_type=jax.ShapeDtypeStruct((NC, NP, D), jnp.float32),
    mesh=plsc.VectorSubcoreMesh(**_MESH),
    scratch_types=[
        pltpu.VMEM((EPT,), jnp.int32),
        pltpu.VMEM((EPT,), jnp.int32),
        pltpu.VMEM((2 * LB, D), jnp.float32),
        pltpu.VMEM_SHARED((NP, D), jnp.float32),
        pltpu.SemaphoreType.DMA((2,)),
    ],
)(_agg_body)


def _edge_body(s1_hbm, s2_hbm, ei_hbm, out_hbm, s_v, srcb_v, dstb_v, outb_v):
    c = lax.axis_index("c")
    s = lax.axis_index("s")
    w = c * NS + s
    base = w * EPT

    pltpu.sync_copy(s1_hbm, s_v.at[pl.ds(0, N)])
    pltpu.sync_copy(s2_hbm, s_v.at[pl.ds(N, N)])
    pltpu.sync_copy(ei_hbm.at[w], srcb_v)
    pltpu.sync_copy(ei_hbm.at[32 + w], dstb_v)

    off2 = jnp.full((16,), N, jnp.int32)

    def step(k, _):
        i1 = srcb_v[pl.ds(k * 16, 16)]
        i2 = dstb_v[pl.ds(k * 16, 16)]
        a = plsc.load_gather(s_v, [i1])
        bb = plsc.load_gather(s_v, [i2 + off2])
        outb_v[pl.ds(k * 16, 16)] = a + bb
        return 0

    lax.fori_loop(0, 625, step, 0)
    pltpu.sync_copy(outb_v, out_hbm.at[pl.ds(base, EPT)])


_edge_call = functools.partial(
    pl.kernel,
    out_type=jax.ShapeDtypeStruct((E,), jnp.float32),
    mesh=plsc.VectorSubcoreMesh(**_MESH),
    compiler_params=pltpu.CompilerParams(needs_layout_passes=False),
    scratch_types=[
        pltpu.VMEM((2 * N,), jnp.float32),
        pltpu.VMEM((EPT,), jnp.int32),
        pltpu.VMEM((EPT,), jnp.int32),
        pltpu.VMEM((EPT,), jnp.float32),
    ],
)(_edge_body)


def _mm_body(x_ref, w_ref, degp_ref, g_ref):
    i = pl.program_id(0)
    deg = degp_ref[0, pl.ds(i * 1280, 1280)] + degp_ref[1, pl.ds(i * 1280, 1280)]
    isd = lax.rsqrt(deg + 1.0)
    h = jnp.dot(x_ref[...], w_ref[...], preferred_element_type=jnp.float32)
    g_ref[...] = h * isd[:, None]


def _mm_call(x, w, degp):
    return pl.pallas_call(
        _mm_body,
        grid=(NP // 1280,),
        in_specs=[
            pl.BlockSpec((1280, D), lambda i: (i, 0)),
            pl.BlockSpec((D, D), lambda i: (0, 0)),
            pl.BlockSpec((NC, NP), lambda i: (0, 0)),
        ],
        out_specs=pl.BlockSpec((1280, D), lambda i: (i, 0)),
        out_shape=jax.ShapeDtypeStruct((N, D), jnp.float32),
    )(x, w, degp)


def _head_body(agg_ref, degp_ref, b_ref, wep_ref, bev_ref, out_ref):
    i = pl.program_id(0)
    deg = degp_ref[0, pl.ds(i * 1280, 1280)] + degp_ref[1, pl.ds(i * 1280, 1280)]
    isd = lax.rsqrt(deg + 1.0)
    agg = agg_ref[0] + agg_ref[1]
    h_out = jnp.maximum(agg * isd[:, None] + b_ref[0], 0.0)
    out_ref[...] = (
        lax.dot_general(wep_ref[...], h_out, (((0,), (1,)), ((), ())),
                        preferred_element_type=jnp.float32)
        + bev_ref[0][:, None]
    )


def _head_call(agg, degp, b2, wep, bev):
    return pl.pallas_call(
        _head_body,
        grid=(NP // 1280,),
        in_specs=[
            pl.BlockSpec((NC, 1280, D), lambda i: (0, i, 0)),
            pl.BlockSpec((NC, NP), lambda i: (0, 0)),
            pl.BlockSpec((1, D), lambda i: (0, 0)),
            pl.BlockSpec((D, 2), lambda i: (0, 0)),
            pl.BlockSpec((1, 2), lambda i: (0, 0)),
        ],
        out_specs=pl.BlockSpec((2, 1280), lambda i: (0, i)),
        out_shape=jax.ShapeDtypeStruct((2, N), jnp.float32),
    )(agg, degp, b2, wep, bev)


def kernel(x, edge_index, W, b, We, be):
    ei = edge_index.astype(jnp.int32).reshape(2 * NC * NS, EPT)

    ones1d = jnp.ones((EPT,), jnp.float32)
    degp = _deg_call(ei, ones1d)
    g = _mm_call(x, W, degp)
    agg = _agg_call(g, ei)

    wep = jnp.zeros((D, 2), jnp.float32)
    wep = wep.at[:, 0].set(We[:D, 0]).at[:, 1].set(We[D:, 0])
    bev = jnp.zeros((1, 2), jnp.float32).at[0, 0].set(be[0])
    s12 = _head_call(agg, degp, b.reshape(1, D), wep, bev)

    eout = _edge_call(s12[0], s12[1], ei)
    return eout[:, None]
